# baseline (speedup 1.0000x reference)
import jax
import jax.numpy as jnp
from jax.experimental import pallas as pl

L_N = 2


def _mlp(ps, x):
    for k, (W, b) in enumerate(ps):
        x = x @ W + b
        if k < len(ps) - 1:
            x = jax.nn.relu(x)
    return x


def _gmp(params, h, g, pos):
    src, dst = g[0], g[1]
    rel = pos[src] - pos[dst]
    dist = jnp.sqrt(jnp.sum(rel * rel, axis=-1, keepdims=True) + 1e-12)
    ef = jnp.concatenate([h[src], h[dst], rel, dist], axis=-1)
    msg = _mlp(params["edge"], ef)
    aggr = jax.ops.segment_sum(msg, dst, num_segments=h.shape[0])
    upd = _mlp(params["node"], jnp.concatenate([h, aggr], axis=-1))
    return h + upd


def _cal_ew(w, g):
    src, dst = g[0], g[1]
    n = w.shape[0]
    deg = jax.ops.segment_sum(jnp.ones(src.shape, jnp.float32), src, num_segments=n)
    deg = jnp.maximum(deg, 1.0)
    normed_w = w[:, 0] / deg
    w_to_send = normed_w[src]
    aggr_w = jax.ops.segment_sum(w_to_send, dst, num_segments=n) + 1e-12
    ec = w_to_send / aggr_w[dst]
    return ec, aggr_w[:, None]


def _edge_conv(x, g, ew, aggragating=True):
    src, dst = g[0], g[1]
    n = x.shape[0]
    if aggragating:
        return jax.ops.segment_sum(x[src] * ew[:, None], dst, num_segments=n)
    return jax.ops.segment_sum(x[dst] * ew[:, None], src, num_segments=n)


def _unpool(h, n, idx):
    return jnp.zeros((n, h.shape[-1]), h.dtype).at[idx].set(h)


def _identity_pallas(x):
    def body(x_ref, o_ref):
        o_ref[...] = x_ref[...]
    return pl.pallas_call(
        body, out_shape=jax.ShapeDtypeStruct(x.shape, x.dtype))(x)


def kernel(h, pos, params, m_ids_0, m_ids_1, m_gs_0, m_gs_1, m_gs_2):
    m_ids = [m_ids_0, m_ids_1]
    m_gs = [m_gs_0, m_gs_1, m_gs_2]
    down_outs, down_ps, cts = [], [], []
    w = jnp.ones((pos.shape[-2], 1), jnp.float32)
    for i in range(L_N):
        h = _gmp(params["down"][i], h, m_gs[i], pos)
        down_outs.append(h)
        down_ps.append(pos)
        tmp_g = m_gs[i]
        ew, w = _cal_ew(w, tmp_g)
        h = _edge_conv(h, tmp_g, ew)
        pos = _edge_conv(pos, tmp_g, ew)
        cts.append(ew)
        h = h[m_ids[i]]
        pos = pos[m_ids[i]]
        w = w[m_ids[i]]
    h = _gmp(params["bottom"], h, m_gs[L_N], pos)
    for i in range(L_N):
        up = L_N - i - 1
        g, idx = m_gs[up], m_ids[up]
        h = _unpool(h, down_outs[up].shape[-2], idx)
        h = _edge_conv(h, g, cts[up], aggragating=False)
        h = _gmp(params["up"][i], h, g, down_ps[up])
        h = h + down_outs[up]
    return _identity_pallas(h)


# speedup vs baseline: 1.0609x; 1.0609x over previous
import functools

import jax
import jax.numpy as jnp
from jax import lax
from jax.experimental import pallas as pl
from jax.experimental.pallas import tpu as pltpu
from jax.experimental.pallas import tpu_sc as plsc

F32 = jnp.float32
I32 = jnp.int32
NCORE = 2
NSUB = 16
NWORK = NCORE * NSUB
CH = 128
LD = 128


def _rup(x, m):
    return (x + m - 1) // m * m


def _mesh():
    return plsc.VectorSubcoreMesh(core_axis_name="c", subcore_axis_name="s")


def _wid():
    return lax.axis_index("s") * NCORE + lax.axis_index("c")


def _chunk(per_w):
    c = per_w
    while c > CH:
        c //= 2
    return c


@functools.lru_cache(maxsize=None)
def _sc_gather2(EP, NP):
    per_w = EP // NWORK
    nch = per_w // CH

    @functools.partial(
        pl.kernel,
        out_type=(jax.ShapeDtypeStruct((EP, LD), F32),
                  jax.ShapeDtypeStruct((EP, LD), F32),
                  jax.ShapeDtypeStruct((EP,), F32)),
        mesh=_mesh(),
        compiler_params=pltpu.CompilerParams(needs_layout_passes=False),
        scratch_types=[pltpu.VMEM((CH,), I32), pltpu.VMEM((CH,), I32),
                       pltpu.VMEM((CH, LD), F32), pltpu.VMEM((CH, LD), F32),
                       pltpu.VMEM((CH,), F32),
                       pltpu.VMEM((NP,), F32), pltpu.VMEM((NP,), F32),
                       pltpu.VMEM((NP,), F32),
                       pltpu.SemaphoreType.DMA, pltpu.SemaphoreType.DMA],
    )
    def k(ta, tb, posx, posy, posz, src, dst, gs, gd, d2o,
          ia, ib, ra, rb, d2v, px, py, pz, s1, s2):
        pltpu.sync_copy(posx, px)
        pltpu.sync_copy(posy, py)
        pltpu.sync_copy(posz, pz)
        base0 = _wid() * per_w

        def body(c, carry):
            b = base0 + c * CH
            pltpu.sync_copy(src.at[pl.ds(b, CH)], ia)
            pltpu.sync_copy(dst.at[pl.ds(b, CH)], ib)
            cp1 = pltpu.async_copy(ta.at[ia], ra, s1)
            cp2 = pltpu.async_copy(tb.at[ib], rb, s2)
            for v in range(CH // 16):
                ivs = ia[pl.ds(v * 16, 16)]
                ivd = ib[pl.ds(v * 16, 16)]
                dx = plsc.load_gather(px, [ivs]) - plsc.load_gather(px, [ivd])
                dy = plsc.load_gather(py, [ivs]) - plsc.load_gather(py, [ivd])
                dz = plsc.load_gather(pz, [ivs]) - plsc.load_gather(pz, [ivd])
                d2v[pl.ds(v * 16, 16)] = dx * dx + dy * dy + dz * dz
            cp1.wait()
            cp2.wait()
            pltpu.sync_copy(ra, gs.at[pl.ds(b, CH)])
            pltpu.sync_copy(rb, gd.at[pl.ds(b, CH)])
            pltpu.sync_copy(d2v, d2o.at[pl.ds(b, CH)])
            return carry

        lax.fori_loop(0, nch, body, 0)

    return k


@functools.lru_cache(maxsize=None)
def _sc_select(MP, W):
    per_w = MP // NWORK
    c = _chunk(per_w)
    nch = per_w // c

    @functools.partial(
        pl.kernel,
        out_type=jax.ShapeDtypeStruct((MP, W), F32),
        mesh=_mesh(),
        compiler_params=pltpu.CompilerParams(needs_layout_passes=False),
        scratch_types=[pltpu.VMEM((c,), I32),
                       pltpu.VMEM((c, W), F32), pltpu.VMEM((c, W), F32),
                       pltpu.SemaphoreType.DMA, pltpu.SemaphoreType.DMA],
    )
    def k(p0, p1, idx, out, iv, ra, rb, s1, s2):
        base0 = _wid() * per_w

        def body(ci, carry):
            b = base0 + ci * c
            pltpu.sync_copy(idx.at[pl.ds(b, c)], iv)
            cp1 = pltpu.async_copy(p0.at[iv], ra, s1)
            cp2 = pltpu.async_copy(p1.at[iv], rb, s2)
            cp1.wait()
            cp2.wait()

            def addrow(r, carry2):
                for q in range(W // 16):
                    ra[r, pl.ds(q * 16, 16)] = (ra[r, pl.ds(q * 16, 16)] +
                                                rb[r, pl.ds(q * 16, 16)])
                return carry2

            lax.fori_loop(0, c, addrow, 0)
            pltpu.sync_copy(ra, out.at[pl.ds(b, c)])
            return carry

        lax.fori_loop(0, nch, body, 0)

    return k


def _zero_accum(zeros_hbm, accum, rpt):
    sid = lax.axis_index("s")
    pltpu.sync_copy(zeros_hbm, accum.at[pl.ds(sid * rpt, rpt)])


def _drain_accum(accum, o0, o1, rpt):
    cid = lax.axis_index("c")
    sid = lax.axis_index("s")

    @pl.when(cid == 0)
    def _():
        pltpu.sync_copy(accum.at[pl.ds(sid * rpt, rpt)],
                        o0.at[pl.ds(sid * rpt, rpt)])

    @pl.when(cid == 1)
    def _():
        pltpu.sync_copy(accum.at[pl.ds(sid * rpt, rpt)],
                        o1.at[pl.ds(sid * rpt, rpt)])


@functools.lru_cache(maxsize=None)
def _sc_scatter_linear(EP, NP):
    per_w = EP // NWORK
    nch = per_w // CH
    rpt = NP // NSUB

    @functools.partial(
        pl.kernel,
        out_type=(jax.ShapeDtypeStruct((NP, LD), F32),
                  jax.ShapeDtypeStruct((NP, LD), F32)),
        mesh=_mesh(),
        compiler_params=pltpu.CompilerParams(needs_layout_passes=False),
        scratch_types=[pltpu.VMEM_SHARED((NP, LD), F32),
                       pltpu.VMEM((CH,), I32), pltpu.VMEM((CH, LD), F32),
                       pltpu.SemaphoreType.DMA],
    )
    def k(t, dst, zeros_hbm, o0, o1, accum, ib, rows, s1):
        _zero_accum(zeros_hbm, accum, rpt)
        plsc.subcore_barrier()
        base0 = _wid() * per_w

        def body(c, carry):
            b = base0 + c * CH
            pltpu.sync_copy(dst.at[pl.ds(b, CH)], ib)
            pltpu.async_copy(t.at[pl.ds(b, CH)], rows, s1).wait()
            pltpu.sync_copy(rows, accum.at[ib], add=True)
            return carry

        lax.fori_loop(0, nch, body, 0)
        plsc.subcore_barrier()
        _drain_accum(accum, o0, o1, rpt)

    return k


@functools.lru_cache(maxsize=None)
def _sc_scatter_gather(EP, NP):
    per_w = EP // NWORK
    nch = per_w // CH
    rpt = NP // NSUB

    @functools.partial(
        pl.kernel,
        out_type=(jax.ShapeDtypeStruct((NP, LD), F32),
                  jax.ShapeDtypeStruct((NP, LD), F32)),
        mesh=_mesh(),
        compiler_params=pltpu.CompilerParams(needs_layout_passes=False),
        scratch_types=[pltpu.VMEM_SHARED((NP, LD), F32),
                       pltpu.VMEM((CH,), I32), pltpu.VMEM((CH,), I32),
                       pltpu.VMEM((CH, LD), F32),
                       pltpu.SemaphoreType.DMA],
    )
    def k(t, src, dst, zeros_hbm, o0, o1, accum, ia, ib, rows, s1):
        _zero_accum(zeros_hbm, accum, rpt)
        plsc.subcore_barrier()
        base0 = _wid() * per_w

        def body(c, carry):
            b = base0 + c * CH
            pltpu.sync_copy(src.at[pl.ds(b, CH)], ia)
            pltpu.sync_copy(dst.at[pl.ds(b, CH)], ib)
            pltpu.async_copy(t.at[ia], rows, s1).wait()
            pltpu.sync_copy(rows, accum.at[ib], add=True)
            return carry

        lax.fori_loop(0, nch, body, 0)
        plsc.subcore_barrier()
        _drain_accum(accum, o0, o1, rpt)

    return k


@functools.lru_cache(maxsize=None)
def _sc_scatter_unpool(EP, NPF, NCP, NC):
    per_w = EP // NWORK
    nch = per_w // CH
    rpt = NPF // NSUB
    NC_IT = NCP // 16

    @functools.partial(
        pl.kernel,
        out_type=(jax.ShapeDtypeStruct((NPF, LD), F32),
                  jax.ShapeDtypeStruct((NPF, LD), F32)),
        mesh=_mesh(),
        compiler_params=pltpu.CompilerParams(needs_layout_passes=False),
        scratch_types=[pltpu.VMEM_SHARED((NPF, LD), F32),
                       pltpu.VMEM_SHARED((NPF,), I32),
                       pltpu.VMEM((NPF,), I32),
                       pltpu.VMEM((NCP,), I32), pltpu.VMEM((NCP,), I32),
                       pltpu.VMEM((CH,), I32), pltpu.VMEM((CH,), I32),
                       pltpu.VMEM((CH,), I32),
                       pltpu.VMEM((CH, LD), F32),
                       pltpu.SemaphoreType.DMA],
    )
    def k(t, ids, ids_next, gidx, sidx, zeros_hbm, o0, o1,
          accum, inv_sh, inv, ids_v, idsn_v, ia, ib, jb, rows, s1):
        _zero_accum(zeros_hbm, accum, rpt)
        sid = lax.axis_index("s")

        @pl.when(sid == 0)
        def _():
            fill = jnp.full((16,), NC, I32)

            def initf(i, carry):
                inv[pl.ds(i * 16, 16)] = fill
                return carry

            lax.fori_loop(0, NPF // 16, initf, 0)
            pltpu.sync_copy(ids, ids_v)
            pltpu.sync_copy(ids_next, idsn_v)
            iota = lax.iota(I32, 16)

            def scan(j, carry):
                cur = ids_v[pl.ds(j * 16, 16)]
                nxt = idsn_v[pl.ds(j * 16, 16)]
                mask = (cur != nxt) & (cur >= 0)
                jvec = iota + j * 16
                plsc.store_scatter(inv, [cur], jvec, mask=mask)
                return carry

            lax.fori_loop(0, NC_IT, scan, 0)
            pltpu.sync_copy(inv, inv_sh)

        plsc.subcore_barrier()
        pltpu.sync_copy(inv_sh, inv)
        plsc.subcore_barrier()

        base0 = _wid() * per_w

        def body(c, carry):
            b = base0 + c * CH
            pltpu.sync_copy(gidx.at[pl.ds(b, CH)], ia)
            pltpu.sync_copy(sidx.at[pl.ds(b, CH)], ib)
            for v in range(CH // 16):
                dv = ia[pl.ds(v * 16, 16)]
                jb[pl.ds(v * 16, 16)] = plsc.load_gather(inv, [dv])
            pltpu.async_copy(t.at[jb], rows, s1).wait()
            pltpu.sync_copy(rows, accum.at[ib], add=True)
            return carry

        lax.fori_loop(0, nch, body, 0)
        plsc.subcore_barrier()
        _drain_accum(accum, o0, o1, rpt)

    return k


@functools.lru_cache(maxsize=None)
def _sc_degree(EP, NP):
    per_w = EP // NWORK
    nch = per_w // CH
    rpt = NP // NSUB

    @functools.partial(
        pl.kernel,
        out_type=(jax.ShapeDtypeStruct((NP,), F32),
                  jax.ShapeDtypeStruct((NP,), F32)),
        mesh=_mesh(),
        compiler_params=pltpu.CompilerParams(needs_layout_passes=False),
        scratch_types=[pltpu.VMEM_SHARED((NSUB * NP,), F32),
                       pltpu.VMEM((NP,), F32), pltpu.VMEM((rpt,), F32),
                       pltpu.VMEM((CH,), I32),
                       pltpu.SemaphoreType.DMA],
    )
    def k(src, o0, o1, part_sh, degv, tmp, ib, s1):
        zero16 = jnp.zeros((16,), F32)

        def zf(i, carry):
            degv[pl.ds(i * 16, 16)] = zero16
            return carry

        lax.fori_loop(0, NP // 16, zf, 0)
        ones = jnp.ones((16,), F32)
        base0 = _wid() * per_w

        def body(c, carry):
            b = base0 + c * CH
            pltpu.sync_copy(src.at[pl.ds(b, CH)], ib)
            for v in range(CH // 16):
                iv = ib[pl.ds(v * 16, 16)]
                plsc.addupdate_scatter(degv, [iv], ones)
            return carry

        lax.fori_loop(0, nch, body, 0)
        sid = lax.axis_index("s")
        pltpu.sync_copy(degv, part_sh.at[pl.ds(sid * NP, NP)])
        plsc.subcore_barrier()
        base = sid * rpt
        pltpu.sync_copy(part_sh.at[pl.ds(base, rpt)], tmp)

        def red(t, carry):
            pltpu.sync_copy(part_sh.at[pl.ds(t * NP + base, rpt)],
                            degv.at[pl.ds(0, rpt)])

            def addv(i, carry2):
                tmp[pl.ds(i * 16, 16)] = (tmp[pl.ds(i * 16, 16)] +
                                          degv[pl.ds(i * 16, 16)])
                return carry2

            lax.fori_loop(0, rpt // 16, addv, 0)
            return carry

        lax.fori_loop(1, NSUB, red, 0)
        cid = lax.axis_index("c")

        @pl.when(cid == 0)
        def _():
            pltpu.sync_copy(tmp, o0.at[pl.ds(base, rpt)])

        @pl.when(cid == 1)
        def _():
            pltpu.sync_copy(tmp, o1.at[pl.ds(base, rpt)])

    return k


BN = 256
BE = 512


@functools.lru_cache(maxsize=None)
def _tc_ab(NP):
    def body(xh_a, xh_b, pos8, sh, wha, whb, wp8, a_out, b_out):
        xh = (xh_a[...] + xh_b[...]) * sh[...]
        pc = jnp.dot(pos8[...], wp8[...], preferred_element_type=F32)
        a = jnp.dot(xh, wha[...], preferred_element_type=F32) + pc
        b = jnp.dot(xh, whb[...], preferred_element_type=F32) - pc
        a_out[...] = a
        b_out[...] = b

    blk = lambda i: (i, 0)
    cst = lambda i: (0, 0)
    return pl.pallas_call(
        body,
        grid=(NP // BN,),
        in_specs=[pl.BlockSpec((BN, LD), blk), pl.BlockSpec((BN, LD), blk),
                  pl.BlockSpec((BN, 8), blk),
                  pl.BlockSpec((BN, 1), blk),
                  pl.BlockSpec((LD, LD), cst), pl.BlockSpec((LD, LD), cst),
                  pl.BlockSpec((8, LD), cst)],
        out_specs=(pl.BlockSpec((BN, LD), blk), pl.BlockSpec((BN, LD), blk)),
        out_shape=(jax.ShapeDtypeStruct((NP, LD), F32),
                   jax.ShapeDtypeStruct((NP, LD), F32)),
    )


@functools.lru_cache(maxsize=None)
def _tc_edge(EP):
    def body(gs, gd, d2, w0d, b0, w1, b1, w2, b2, out):
        dist = jnp.sqrt(d2[...] + 1e-12)
        z = gs[...] + gd[...] + dist * w0d[...] + b0[...]
        y = jnp.maximum(z, 0.0)
        y = jnp.dot(y, w1[...], preferred_element_type=F32) + b1[...]
        y = jnp.maximum(y, 0.0)
        out[...] = jnp.dot(y, w2[...], preferred_element_type=F32) + b2[...]

    blk = lambda i: (i, 0)
    cst = lambda i: (0, 0)
    return pl.pallas_call(
        body,
        grid=(EP // BE,),
        in_specs=[pl.BlockSpec((BE, LD), blk), pl.BlockSpec((BE, LD), blk),
                  pl.BlockSpec((BE, 1), blk),
                  pl.BlockSpec((1, LD), cst), pl.BlockSpec((1, LD), cst),
                  pl.BlockSpec((LD, LD), cst), pl.BlockSpec((1, LD), cst),
                  pl.BlockSpec((LD, LD), cst), pl.BlockSpec((1, LD), cst)],
        out_specs=pl.BlockSpec((BE, LD), blk),
        out_shape=jax.ShapeDtypeStruct((EP, LD), F32),
    )


@functools.lru_cache(maxsize=None)
def _tc_node(NP, want_y, want_div, want_skip):
    def body(*refs):
        it = iter(refs)
        ha, hb, sh, p0, p1 = (next(it) for _ in range(5))
        wn0a, wn0b, bn0, wn1, bn1, wn2, bn2 = (next(it) for _ in range(7))
        skip = next(it) if want_skip else None
        if want_y:
            pos8, nwv = next(it), next(it)
        sdiv = next(it) if want_div else None
        hout = next(it)
        if want_y:
            yout, auxout = next(it), next(it)
        dout = next(it) if want_div else None

        he = (ha[...] + hb[...]) * sh[...]
        ag = p0[...] + p1[...]
        z = (jnp.dot(he, wn0a[...], preferred_element_type=F32) +
             jnp.dot(ag, wn0b[...], preferred_element_type=F32) + bn0[...])
        y = jnp.maximum(z, 0.0)
        y = jnp.dot(y, wn1[...], preferred_element_type=F32) + bn1[...]
        y = jnp.maximum(y, 0.0)
        u = jnp.dot(y, wn2[...], preferred_element_type=F32) + bn2[...]
        ho = he + u
        if want_skip:
            ho = ho + skip[...]
        hout[...] = ho
        if want_y:
            nw = nwv[...]
            yout[...] = ho * nw
            auxout[...] = jnp.concatenate(
                [pos8[...] * nw, nw, jnp.zeros((BN, 119), F32)], axis=1)
        if want_div:
            dout[...] = ho * sdiv[...]

    blk = lambda i: (i, 0)
    cst = lambda i: (0, 0)
    in_specs = [pl.BlockSpec((BN, LD), blk), pl.BlockSpec((BN, LD), blk),
                pl.BlockSpec((BN, 1), blk),
                pl.BlockSpec((BN, LD), blk), pl.BlockSpec((BN, LD), blk),
                pl.BlockSpec((LD, LD), cst), pl.BlockSpec((LD, LD), cst),
                pl.BlockSpec((1, LD), cst),
                pl.BlockSpec((LD, LD), cst), pl.BlockSpec((1, LD), cst),
                pl.BlockSpec((LD, LD), cst), pl.BlockSpec((1, LD), cst)]
    if want_skip:
        in_specs.append(pl.BlockSpec((BN, LD), blk))
    if want_y:
        in_specs += [pl.BlockSpec((BN, 8), blk), pl.BlockSpec((BN, 1), blk)]
    if want_div:
        in_specs.append(pl.BlockSpec((BN, 1), blk))
    out_specs = [pl.BlockSpec((BN, LD), blk)]
    out_shape = [jax.ShapeDtypeStruct((NP, LD), F32)]
    if want_y:
        out_specs += [pl.BlockSpec((BN, LD), blk), pl.BlockSpec((BN, LD), blk)]
        out_shape += [jax.ShapeDtypeStruct((NP, LD), F32),
                      jax.ShapeDtypeStruct((NP, LD), F32)]
    if want_div:
        out_specs.append(pl.BlockSpec((BN, LD), blk))
        out_shape.append(jax.ShapeDtypeStruct((NP, LD), F32))
    return pl.pallas_call(
        body,
        grid=(NP // BN,),
        in_specs=in_specs,
        out_specs=tuple(out_specs),
        out_shape=tuple(out_shape),
    )


def _prep_gmp(p):
    w0, b0 = p["edge"][0]
    w1, b1 = p["edge"][1]
    w2, b2 = p["edge"][2]
    wp8 = jnp.concatenate([w0[256:259], jnp.zeros((5, 128), F32)], axis=0)
    wn0, bn0 = p["node"][0]
    wn1, bn1 = p["node"][1]
    wn2, bn2 = p["node"][2]
    return dict(wha=w0[:128], whb=w0[128:256], wp8=wp8,
                w0d=w0[259:260], b0=b0[None, :],
                w1=w1, b1=b1[None, :], w2=w2, b2=b2[None, :],
                wn0a=wn0[:128], wn0b=wn0[128:], bn0=bn0[None, :],
                wn1=wn1, bn1=bn1[None, :], wn2=wn2, bn2=bn2[None, :])


def _pad_rows(x, np_rows):
    return jnp.pad(x, ((0, np_rows - x.shape[0]), (0, 0)))


def _gmp_core(wp, ha, hb, sh, pos8, srcP, dstP, EP, NP,
              zeros128, want_y=False, want_div=False, skip=None,
              nwv=None, sdiv=None):
    a_t, b_t = _tc_ab(NP)(ha, hb, pos8, sh,
                          wp["wha"], wp["whb"], wp["wp8"])
    posx, posy, posz = pos8[:, 0], pos8[:, 1], pos8[:, 2]
    gs, gd, d2 = _sc_gather2(EP, NP)(a_t, b_t, posx, posy, posz, srcP, dstP)
    msg = _tc_edge(EP)(gs, gd, d2[:, None], wp["w0d"], wp["b0"],
                       wp["w1"], wp["b1"], wp["w2"], wp["b2"])
    ag0, ag1 = _sc_scatter_linear(EP, NP)(msg, dstP, zeros128)
    args = [ha, hb, sh, ag0, ag1, wp["wn0a"], wp["wn0b"], wp["bn0"],
            wp["wn1"], wp["bn1"], wp["wn2"], wp["bn2"]]
    if skip is not None:
        args.append(skip)
    if want_y:
        args += [pos8, nwv]
    if want_div:
        args.append(sdiv)
    return _tc_node(NP, want_y, want_div, skip is not None)(*args)


def kernel(h, pos, params, m_ids_0, m_ids_1, m_gs_0, m_gs_1, m_gs_2):
    m_ids = [m_ids_0, m_ids_1]
    m_gs = [m_gs_0, m_gs_1, m_gs_2]
    NS = [h.shape[0], m_ids_0.shape[0], m_ids_1.shape[0]]
    NP = [_rup(n + 8, 256) for n in NS]
    EP = [_rup(g.shape[1], NWORK * CH) for g in m_gs]

    srcP, dstP = [], []
    for l in range(3):
        g = m_gs[l]
        padv = jnp.full((EP[l] - g.shape[1],), NS[l], I32)
        srcP.append(jnp.concatenate([g[0], padv]))
        dstP.append(jnp.concatenate([g[1], padv]))

    zeros = {n: jnp.zeros((n // NSUB, LD), F32) for n in set(NP)}

    wps = {"down": [_prep_gmp(p) for p in params["down"]],
           "up": [_prep_gmp(p) for p in params["up"]],
           "bottom": _prep_gmp(params["bottom"])}

    onesv = [jnp.ones((n, 1), F32) for n in NP]
    z128 = [jnp.zeros((n, LD), F32) for n in NP]

    ha, hb = _pad_rows(h, NP[0]), z128[0]
    pos8 = _pad_rows(jnp.pad(pos, ((0, 0), (0, 5))), NP[0])
    sh = onesv[0]
    w = jnp.ones((NP[0], 1), F32)

    down_outs, down_pos8, nws, wins = [], [], [], []
    for i in range(2):
        NPi, EPi = NP[i], EP[i]
        d0, d1 = _sc_degree(EPi, NPi)(srcP[i])
        deg = jnp.maximum((d0 + d1)[:, None], 1.0)
        nwv = w / deg
        hout, Y, Yaux = _gmp_core(wps["down"][i], ha, hb, sh, pos8,
                                  srcP[i], dstP[i], EPi, NPi, zeros[NPi],
                                  want_y=True, nwv=nwv)
        down_outs.append(hout)
        down_pos8.append(pos8)
        nws.append(nwv)
        wins.append(w)
        p0, p1 = _sc_scatter_gather(EPi, NPi)(Y, srcP[i], dstP[i], zeros[NPi])
        q0, q1 = _sc_scatter_gather(EPi, NPi)(Yaux, srcP[i], dstP[i],
                                              zeros[NPi])
        midP = jnp.pad(m_ids[i], (0, NP[i + 1] - NS[i + 1]))
        x2h = _sc_select(NP[i + 1], LD)(p0, p1, midP)
        x2a = _sc_select(NP[i + 1], LD)(q0, q1, midP)
        aw2 = x2a[:, 8:9] + 1e-12
        inv_aw = 1.0 / aw2
        ha, hb = x2h, z128[i + 1]
        sh = inv_aw
        pos8 = x2a[:, :8] * inv_aw
        w = aw2

    hbot, hdiv = _gmp_core(wps["bottom"], ha, hb, sh, pos8,
                           srcP[2], dstP[2], EP[2], NP[2], zeros[NP[2]],
                           want_div=True, sdiv=1.0 / w)

    hcur = hbot
    for i in range(2):
        up = 1 - i
        NPf, NPc, EPu = NP[up], NP[up + 1], EP[up]
        nc = NS[up + 1]
        ncp = _rup(nc, CH)
        ids = jnp.pad(m_ids[up], (0, ncp - nc), constant_values=-8)
        ids_next = jnp.pad(m_ids[up][1:], (0, ncp - nc + 1),
                           constant_values=-9)
        rowid = lax.broadcasted_iota(I32, (NPc, 1), 0)
        zc = jnp.where(rowid < nc, hdiv, 0.0)
        c0, c1 = _sc_scatter_unpool(EPu, NPf, ncp, nc)(
            zc, ids, ids_next, dstP[up], srcP[up], zeros[NPf])
        want_div = (i == 0)
        outs = _gmp_core(wps["up"][i], c0, c1, nws[up],
                         down_pos8[up], srcP[up], dstP[up], EPu, NPf,
                         zeros[NPf], want_div=want_div, skip=down_outs[up],
                         sdiv=(1.0 / wins[up]) if want_div else None)
        if want_div:
            hcur, hdiv = outs
        else:
            (hcur,) = outs

    return hcur[:NS[0]]


# speedup vs baseline: 1.0688x; 1.0075x over previous
import functools

import jax
import jax.numpy as jnp
from jax import lax
from jax.experimental import pallas as pl
from jax.experimental.pallas import tpu as pltpu
from jax.experimental.pallas import tpu_sc as plsc

F32 = jnp.float32
I32 = jnp.int32
NCORE = 2
NSUB = 16
NWORK = NCORE * NSUB
CH = 128
RING = 4
LD = 128

_SC_PARAMS = dict(
    compiler_params=pltpu.CompilerParams(needs_layout_passes=False))


def _rup(x, m):
    return (x + m - 1) // m * m


def _mesh():
    return plsc.VectorSubcoreMesh(core_axis_name="c", subcore_axis_name="s")


def _wid():
    return lax.axis_index("s") * NCORE + lax.axis_index("c")


@functools.lru_cache(maxsize=None)
def _sc_gather2(EP, NP):
    per_w = EP // NWORK
    nch = per_w // CH
    R = 2
    ng = nch // R

    @functools.partial(
        pl.kernel,
        out_type=(jax.ShapeDtypeStruct((EP, LD), F32),
                  jax.ShapeDtypeStruct((EP, LD), F32),
                  jax.ShapeDtypeStruct((EP,), F32)),
        mesh=_mesh(),
        scratch_types=[pltpu.VMEM((CH,), I32), pltpu.VMEM((CH,), I32),
                       pltpu.VMEM((CH,), I32), pltpu.VMEM((CH,), I32),
                       pltpu.VMEM((CH, LD), F32), pltpu.VMEM((CH, LD), F32),
                       pltpu.VMEM((CH, LD), F32), pltpu.VMEM((CH, LD), F32),
                       pltpu.VMEM((CH,), F32),
                       pltpu.VMEM((NP,), F32), pltpu.VMEM((NP,), F32),
                       pltpu.VMEM((NP,), F32)]
        + [pltpu.SemaphoreType.DMA] * 4,
        **_SC_PARAMS,
    )
    def k(ta, tb, posx, posy, posz, src, dst, gs, gd, d2o,
          ia0, ia1, ib0, ib1, ra0, ra1, rb0, rb1, d2v, px, py, pz,
          s0, s1, s2, s3):
        ias = (ia0, ia1)
        ibs = (ib0, ib1)
        ras = (ra0, ra1)
        rbs = (rb0, rb1)
        sas = (s0, s1)
        sbs = (s2, s3)
        wid = _wid()
        base0 = wid * per_w
        pltpu.sync_copy(posx, px)
        pltpu.sync_copy(posy, py)
        pltpu.sync_copy(posz, pz)

        for b in range(R):
            pltpu.sync_copy(src.at[pl.ds(base0 + b * CH, CH)], ias[b])
            pltpu.sync_copy(dst.at[pl.ds(base0 + b * CH, CH)], ibs[b])
            pltpu.async_copy(ta.at[ias[b]], ras[b], sas[b])
            pltpu.async_copy(tb.at[ibs[b]], rbs[b], sbs[b])

        def outer(gi, carry):
            for b in range(R):
                c = gi * R + b
                e0 = base0 + c * CH
                for v in range(CH // 16):
                    ivs = ias[b][pl.ds(v * 16, 16)]
                    ivd = ibs[b][pl.ds(v * 16, 16)]
                    dx = (plsc.load_gather(px, [ivs]) -
                          plsc.load_gather(px, [ivd]))
                    dy = (plsc.load_gather(py, [ivs]) -
                          plsc.load_gather(py, [ivd]))
                    dz = (plsc.load_gather(pz, [ivs]) -
                          plsc.load_gather(pz, [ivd]))
                    d2v[pl.ds(v * 16, 16)] = dx * dx + dy * dy + dz * dz
                pltpu.sync_copy(d2v, d2o.at[pl.ds(e0, CH)])
                pltpu.make_async_copy(ta.at[ias[b]], ras[b], sas[b]).wait()
                pltpu.sync_copy(ras[b], gs.at[pl.ds(e0, CH)])
                pltpu.make_async_copy(tb.at[ibs[b]], rbs[b], sbs[b]).wait()
                pltpu.sync_copy(rbs[b], gd.at[pl.ds(e0, CH)])
                nxt = c + R

                @pl.when(nxt < nch)
                def _():
                    e1 = base0 + nxt * CH
                    pltpu.sync_copy(src.at[pl.ds(e1, CH)], ias[b])
                    pltpu.sync_copy(dst.at[pl.ds(e1, CH)], ibs[b])
                    pltpu.async_copy(ta.at[ias[b]], ras[b], sas[b])
                    pltpu.async_copy(tb.at[ibs[b]], rbs[b], sbs[b])

            return carry

        lax.fori_loop(0, ng, outer, 0)

    return k


def _zero_accum(zeros_hbm, accum, rpt):
    sid = lax.axis_index("s")
    pltpu.sync_copy(zeros_hbm, accum.at[pl.ds(sid * rpt, rpt)])


def _drain_accum(accum, o0, o1, rpt):
    cid = lax.axis_index("c")
    sid = lax.axis_index("s")

    @pl.when(cid == 0)
    def _():
        pltpu.sync_copy(accum.at[pl.ds(sid * rpt, rpt)],
                        o0.at[pl.ds(sid * rpt, rpt)])

    @pl.when(cid == 1)
    def _():
        pltpu.sync_copy(accum.at[pl.ds(sid * rpt, rpt)],
                        o1.at[pl.ds(sid * rpt, rpt)])


@functools.lru_cache(maxsize=None)
def _sc_scatter_linear(EP, NP):
    per_w = EP // NWORK
    nch = per_w // CH
    R = 2
    ng = nch // R
    rpt = NP // NSUB

    @functools.partial(
        pl.kernel,
        out_type=(jax.ShapeDtypeStruct((NP, LD), F32),
                  jax.ShapeDtypeStruct((NP, LD), F32)),
        mesh=_mesh(),
        scratch_types=[pltpu.VMEM_SHARED((NP, LD), F32),
                       pltpu.VMEM((CH,), I32), pltpu.VMEM((CH,), I32),
                       pltpu.VMEM((CH, LD), F32), pltpu.VMEM((CH, LD), F32),
                       pltpu.SemaphoreType.DMA, pltpu.SemaphoreType.DMA],
        **_SC_PARAMS,
    )
    def k(t, dst, zeros_hbm, o0, o1, accum, ib0, ib1, r0, r1, s0, s1):
        ibs = (ib0, ib1)
        rows = (r0, r1)
        sems = (s0, s1)
        _zero_accum(zeros_hbm, accum, rpt)
        wid = _wid()
        base0 = wid * per_w
        plsc.subcore_barrier()

        for b in range(R):
            e1 = base0 + b * CH
            pltpu.sync_copy(dst.at[pl.ds(e1, CH)], ibs[b])
            pltpu.async_copy(t.at[pl.ds(e1, CH)], rows[b], sems[b])

        def outer(gi, carry):
            for b in range(R):
                c = gi * R + b
                e0 = base0 + c * CH
                pltpu.make_async_copy(t.at[pl.ds(e0, CH)], rows[b],
                                      sems[b]).wait()
                pltpu.sync_copy(rows[b], accum.at[ibs[b]], add=True)
                nxt = c + R

                @pl.when(nxt < nch)
                def _():
                    e1 = base0 + nxt * CH
                    pltpu.sync_copy(dst.at[pl.ds(e1, CH)], ibs[b])
                    pltpu.async_copy(t.at[pl.ds(e1, CH)], rows[b], sems[b])

            return carry

        lax.fori_loop(0, ng, outer, 0)
        plsc.subcore_barrier()
        _drain_accum(accum, o0, o1, rpt)

    return k


@functools.lru_cache(maxsize=None)
def _sc_scatter_gather(EP, NP):
    per_w = EP // NWORK
    nch = per_w // CH
    R = 2
    ng = nch // R
    rpt = NP // NSUB

    @functools.partial(
        pl.kernel,
        out_type=(jax.ShapeDtypeStruct((NP, LD), F32),
                  jax.ShapeDtypeStruct((NP, LD), F32)),
        mesh=_mesh(),
        scratch_types=[pltpu.VMEM_SHARED((NP, LD), F32),
                       pltpu.VMEM((CH,), I32), pltpu.VMEM((CH,), I32),
                       pltpu.VMEM((CH,), I32), pltpu.VMEM((CH,), I32),
                       pltpu.VMEM((CH, LD), F32), pltpu.VMEM((CH, LD), F32),
                       pltpu.SemaphoreType.DMA, pltpu.SemaphoreType.DMA],
        **_SC_PARAMS,
    )
    def k(t, src, dst, zeros_hbm, o0, o1, accum, ia0, ia1, ib0, ib1,
          r0, r1, s0, s1):
        ias = (ia0, ia1)
        ibs = (ib0, ib1)
        rows = (r0, r1)
        sems = (s0, s1)
        _zero_accum(zeros_hbm, accum, rpt)
        wid = _wid()
        base0 = wid * per_w
        plsc.subcore_barrier()

        for b in range(R):
            e1 = base0 + b * CH
            pltpu.sync_copy(src.at[pl.ds(e1, CH)], ias[b])
            pltpu.sync_copy(dst.at[pl.ds(e1, CH)], ibs[b])
            pltpu.async_copy(t.at[ias[b]], rows[b], sems[b])

        def outer(gi, carry):
            for b in range(R):
                c = gi * R + b
                pltpu.make_async_copy(t.at[ias[b]], rows[b], sems[b]).wait()
                pltpu.sync_copy(rows[b], accum.at[ibs[b]], add=True)
                nxt = c + R

                @pl.when(nxt < nch)
                def _():
                    e1 = base0 + nxt * CH
                    pltpu.sync_copy(src.at[pl.ds(e1, CH)], ias[b])
                    pltpu.sync_copy(dst.at[pl.ds(e1, CH)], ibs[b])
                    pltpu.async_copy(t.at[ias[b]], rows[b], sems[b])

            return carry

        lax.fori_loop(0, ng, outer, 0)
        plsc.subcore_barrier()
        _drain_accum(accum, o0, o1, rpt)

    return k


@functools.lru_cache(maxsize=None)
def _sc_translate(EP, NPF, NCP, NC):
    per_w = EP // NWORK
    nch = per_w // CH
    NC_IT = NCP // 16

    @functools.partial(
        pl.kernel,
        out_type=jax.ShapeDtypeStruct((EP,), I32),
        mesh=_mesh(),
        scratch_types=[pltpu.VMEM_SHARED((NPF,), I32),
                       pltpu.VMEM((NPF,), I32),
                       pltpu.VMEM((NCP,), I32), pltpu.VMEM((NCP,), I32),
                       pltpu.VMEM((CH,), I32), pltpu.VMEM((CH,), I32),
                       pltpu.SemaphoreType.DMA],
        **_SC_PARAMS,
    )
    def k(ids, ids_next, dst, jb_out, inv_sh, inv, ids_v, idsn_v,
          ia, jb, s1):
        sid = lax.axis_index("s")

        @pl.when(sid == 0)
        def _():
            fill = jnp.full((16,), NC, I32)

            def initf(i, carry):
                inv[pl.ds(i * 16, 16)] = fill
                return carry

            lax.fori_loop(0, NPF // 16, initf, 0)
            pltpu.sync_copy(ids, ids_v)
            pltpu.sync_copy(ids_next, idsn_v)
            iota = lax.iota(I32, 16)

            def scan(j, carry):
                cur = ids_v[pl.ds(j * 16, 16)]
                nxt = idsn_v[pl.ds(j * 16, 16)]
                mask = (cur != nxt) & (cur >= 0)
                jvec = iota + j * 16
                plsc.store_scatter(inv, [cur], jvec, mask=mask)
                return carry

            lax.fori_loop(0, NC_IT, scan, 0)
            pltpu.sync_copy(inv, inv_sh)

        plsc.subcore_barrier()
        pltpu.sync_copy(inv_sh, inv)
        base0 = _wid() * per_w

        def body(c, carry):
            e0 = base0 + c * CH
            pltpu.sync_copy(dst.at[pl.ds(e0, CH)], ia)
            for v in range(CH // 16):
                dv = ia[pl.ds(v * 16, 16)]
                jb[pl.ds(v * 16, 16)] = plsc.load_gather(inv, [dv])
            pltpu.sync_copy(jb, jb_out.at[pl.ds(e0, CH)])
            return carry

        lax.fori_loop(0, nch, body, 0)

    return k


@functools.lru_cache(maxsize=None)
def _sc_degree(EP, NP):
    per_w = EP // NWORK
    nch = per_w // CH
    rpt = NP // NSUB

    @functools.partial(
        pl.kernel,
        out_type=(jax.ShapeDtypeStruct((NP,), F32),
                  jax.ShapeDtypeStruct((NP,), F32)),
        mesh=_mesh(),
        scratch_types=[pltpu.VMEM_SHARED((NSUB * NP,), F32),
                       pltpu.VMEM((NP,), F32), pltpu.VMEM((rpt,), F32),
                       pltpu.VMEM((nch, CH), I32),
                       pltpu.SemaphoreType.DMA],
        **_SC_PARAMS,
    )
    def k(src, o0, o1, part_sh, degv, tmp, ib, s1):
        zero16 = jnp.zeros((16,), F32)

        def zf(i, carry):
            degv[pl.ds(i * 16, 16)] = zero16
            return carry

        lax.fori_loop(0, NP // 16, zf, 0)
        ones = jnp.ones((16,), F32)
        wid = _wid()
        pltpu.sync_copy(src.at[pl.ds(wid * nch, nch)], ib)

        def body(c, carry):
            for v in range(CH // 16):
                iv = ib[c, pl.ds(v * 16, 16)]
                plsc.addupdate_scatter(degv, [iv], ones)
            return carry

        lax.fori_loop(0, nch, body, 0)
        sid = lax.axis_index("s")
        pltpu.sync_copy(degv, part_sh.at[pl.ds(sid * NP, NP)])
        plsc.subcore_barrier()
        base = sid * rpt
        pltpu.sync_copy(part_sh.at[pl.ds(base, rpt)], tmp)

        def red(t, carry):
            pltpu.sync_copy(part_sh.at[pl.ds(t * NP + base, rpt)],
                            degv.at[pl.ds(0, rpt)])

            def addv(i, carry2):
                tmp[pl.ds(i * 16, 16)] = (tmp[pl.ds(i * 16, 16)] +
                                          degv[pl.ds(i * 16, 16)])
                return carry2

            lax.fori_loop(0, rpt // 16, addv, 0)
            return carry

        lax.fori_loop(1, NSUB, red, 0)
        cid = lax.axis_index("c")

        @pl.when(cid == 0)
        def _():
            pltpu.sync_copy(tmp, o0.at[pl.ds(base, rpt)])

        @pl.when(cid == 1)
        def _():
            pltpu.sync_copy(tmp, o1.at[pl.ds(base, rpt)])

    return k


@functools.lru_cache(maxsize=None)
def _sc_auxpool(EP, NP):
    per_w = EP // NWORK
    nch = per_w // CH
    rpt = NP // NSUB
    NQ = 4

    @functools.partial(
        pl.kernel,
        out_type=tuple(jax.ShapeDtypeStruct((NP,), F32)
                       for _ in range(2 * NQ)),
        mesh=_mesh(),
        scratch_types=[pltpu.VMEM_SHARED((NSUB * NP,), F32)]
        + [pltpu.VMEM((NP,), F32)] * NQ
        + [pltpu.VMEM((NP,), F32)] * NQ
        + [pltpu.VMEM((rpt,), F32), pltpu.VMEM((rpt,), F32),
           pltpu.VMEM((nch, CH), I32), pltpu.VMEM((nch, CH), I32),
           pltpu.SemaphoreType.DMA],
        **_SC_PARAMS,
    )
    def k(qn, qx, qy, qz, src, dst,
          on0, on1, ox0, ox1, oy0, oy1, oz0, oz1,
          part_sh, tn, tx, ty, tz, an, ax, ay, az, tmp, tmp2, ia, ib, s1):
        tabs = (tn, tx, ty, tz)
        accs = (an, ax, ay, az)
        ins = (qn, qx, qy, qz)
        outs = ((on0, on1), (ox0, ox1), (oy0, oy1), (oz0, oz1))
        for q in range(NQ):
            pltpu.sync_copy(ins[q], tabs[q])
        zero16 = jnp.zeros((16,), F32)

        def zf(i, carry):
            for q in range(NQ):
                accs[q][pl.ds(i * 16, 16)] = zero16
            return carry

        lax.fori_loop(0, NP // 16, zf, 0)
        wid = _wid()
        crow0 = wid * nch
        pltpu.sync_copy(src.at[pl.ds(crow0, nch)], ia)
        pltpu.sync_copy(dst.at[pl.ds(crow0, nch)], ib)

        def body(c, carry):
            for v in range(CH // 16):
                sv = ia[c, pl.ds(v * 16, 16)]
                dv = ib[c, pl.ds(v * 16, 16)]
                for q in range(NQ):
                    val = plsc.load_gather(tabs[q], [sv])
                    plsc.addupdate_scatter(accs[q], [dv], val)
            return carry

        lax.fori_loop(0, nch, body, 0)
        sid = lax.axis_index("s")
        cid = lax.axis_index("c")
        base = sid * rpt
        for q in range(NQ):
            pltpu.sync_copy(accs[q], part_sh.at[pl.ds(sid * NP, NP)])
            plsc.subcore_barrier()
            pltpu.sync_copy(part_sh.at[pl.ds(base, rpt)], tmp)

            def red(t, carry):
                pltpu.sync_copy(part_sh.at[pl.ds(t * NP + base, rpt)], tmp2)

                def addv(i, carry2):
                    tmp[pl.ds(i * 16, 16)] = (tmp[pl.ds(i * 16, 16)] +
                                              tmp2[pl.ds(i * 16, 16)])
                    return carry2

                lax.fori_loop(0, rpt // 16, addv, 0)
                return carry

            lax.fori_loop(1, NSUB, red, 0)
            o0, o1 = outs[q]

            @pl.when(cid == 0)
            def _():
                pltpu.sync_copy(tmp, o0.at[pl.ds(base, rpt)])

            @pl.when(cid == 1)
            def _():
                pltpu.sync_copy(tmp, o1.at[pl.ds(base, rpt)])

            plsc.subcore_barrier()

    return k


@functools.lru_cache(maxsize=None)
def _sc_select(MP, NP):
    per_w = MP // NWORK
    c = per_w
    while c > CH:
        c //= 2
    nch = per_w // c

    @functools.partial(
        pl.kernel,
        out_type=(jax.ShapeDtypeStruct((MP, LD), F32),
                  jax.ShapeDtypeStruct((MP,), F32),
                  jax.ShapeDtypeStruct((MP,), F32),
                  jax.ShapeDtypeStruct((MP,), F32),
                  jax.ShapeDtypeStruct((MP,), F32)),
        mesh=_mesh(),
        scratch_types=[pltpu.VMEM((c,), I32),
                       pltpu.VMEM((c, LD), F32), pltpu.VMEM((c, LD), F32)]
        + [pltpu.VMEM((NP,), F32)] * 4
        + [pltpu.VMEM((c,), F32)] * 4
        + [pltpu.SemaphoreType.DMA, pltpu.SemaphoreType.DMA],
        **_SC_PARAMS,
    )
    def k(p0, p1, qn, qx, qy, qz, idx, xo, no, xo2, yo2, zo2,
          iv, ra, rb, tn, tx, ty, tz, vn, vx, vy, vz, s1, s2):
        tabs = (tn, tx, ty, tz)
        vecs = (vn, vx, vy, vz)
        outs = (no, xo2, yo2, zo2)
        ins = (qn, qx, qy, qz)
        for q in range(4):
            pltpu.sync_copy(ins[q], tabs[q])
        base0 = _wid() * per_w

        def body(ci, carry):
            b = base0 + ci * c
            pltpu.sync_copy(idx.at[pl.ds(b, c)], iv)
            cp1 = pltpu.async_copy(p0.at[iv], ra, s1)
            cp2 = pltpu.async_copy(p1.at[iv], rb, s2)
            for v in range(c // 16):
                ivv = iv[pl.ds(v * 16, 16)]
                for q in range(4):
                    vecs[q][pl.ds(v * 16, 16)] = plsc.load_gather(
                        tabs[q], [ivv])
            cp1.wait()
            cp2.wait()

            def addrow(r, carry2):
                for w in range(LD // 16):
                    ra[r, pl.ds(w * 16, 16)] = (ra[r, pl.ds(w * 16, 16)] +
                                                rb[r, pl.ds(w * 16, 16)])
                return carry2

            lax.fori_loop(0, c, addrow, 0)
            pltpu.sync_copy(ra, xo.at[pl.ds(b, c)])
            for q in range(4):
                pltpu.sync_copy(vecs[q], outs[q].at[pl.ds(b, c)])
            return carry

        lax.fori_loop(0, nch, body, 0)

    return k


BN = 256
BE = 512


@functools.lru_cache(maxsize=None)
def _tc_ab(NP):
    def body(xh_a, xh_b, pos8, sh, wha, whb, wp8, a_out, b_out):
        xh = (xh_a[...] + xh_b[...]) * sh[...]
        pc = jnp.dot(pos8[...], wp8[...], preferred_element_type=F32)
        a = jnp.dot(xh, wha[...], preferred_element_type=F32) + pc
        b = jnp.dot(xh, whb[...], preferred_element_type=F32) - pc
        a_out[...] = a
        b_out[...] = b

    blk = lambda i: (i, 0)
    cst = lambda i: (0, 0)
    return pl.pallas_call(
        body,
        grid=(NP // BN,),
        in_specs=[pl.BlockSpec((BN, LD), blk), pl.BlockSpec((BN, LD), blk),
                  pl.BlockSpec((BN, 8), blk),
                  pl.BlockSpec((BN, 1), blk),
                  pl.BlockSpec((LD, LD), cst), pl.BlockSpec((LD, LD), cst),
                  pl.BlockSpec((8, LD), cst)],
        out_specs=(pl.BlockSpec((BN, LD), blk), pl.BlockSpec((BN, LD), blk)),
        out_shape=(jax.ShapeDtypeStruct((NP, LD), F32),
                   jax.ShapeDtypeStruct((NP, LD), F32)),
    )


@functools.lru_cache(maxsize=None)
def _tc_edge(EP):
    def body(gs, gd, d2, w0d, b0, w1, b1, w2, b2, out):
        dist = jnp.sqrt(d2[...] + 1e-12)
        z = gs[...] + gd[...] + dist * w0d[...] + b0[...]
        y = jnp.maximum(z, 0.0)
        y = jnp.dot(y, w1[...], preferred_element_type=F32) + b1[...]
        y = jnp.maximum(y, 0.0)
        out[...] = jnp.dot(y, w2[...], preferred_element_type=F32) + b2[...]

    blk = lambda i: (i, 0)
    cst = lambda i: (0, 0)
    return pl.pallas_call(
        body,
        grid=(EP // BE,),
        in_specs=[pl.BlockSpec((BE, LD), blk), pl.BlockSpec((BE, LD), blk),
                  pl.BlockSpec((BE, 1), blk),
                  pl.BlockSpec((1, LD), cst), pl.BlockSpec((1, LD), cst),
                  pl.BlockSpec((LD, LD), cst), pl.BlockSpec((1, LD), cst),
                  pl.BlockSpec((LD, LD), cst), pl.BlockSpec((1, LD), cst)],
        out_specs=pl.BlockSpec((BE, LD), blk),
        out_shape=jax.ShapeDtypeStruct((EP, LD), F32),
    )


@functools.lru_cache(maxsize=None)
def _tc_node(NP, want_y, want_div, want_skip):
    def body(*refs):
        it = iter(refs)
        ha, hb, sh, p0, p1 = (next(it) for _ in range(5))
        wn0a, wn0b, bn0, wn1, bn1, wn2, bn2 = (next(it) for _ in range(7))
        skip = next(it) if want_skip else None
        nwv = next(it) if want_y else None
        sdiv = next(it) if want_div else None
        hout = next(it)
        yout = next(it) if want_y else None
        dout = next(it) if want_div else None

        he = (ha[...] + hb[...]) * sh[...]
        ag = p0[...] + p1[...]
        z = (jnp.dot(he, wn0a[...], preferred_element_type=F32) +
             jnp.dot(ag, wn0b[...], preferred_element_type=F32) + bn0[...])
        y = jnp.maximum(z, 0.0)
        y = jnp.dot(y, wn1[...], preferred_element_type=F32) + bn1[...]
        y = jnp.maximum(y, 0.0)
        u = jnp.dot(y, wn2[...], preferred_element_type=F32) + bn2[...]
        ho = he + u
        if want_skip:
            ho = ho + skip[...]
        hout[...] = ho
        if want_y:
            yout[...] = ho * nwv[...]
        if want_div:
            dout[...] = ho * sdiv[...]

    blk = lambda i: (i, 0)
    cst = lambda i: (0, 0)
    in_specs = [pl.BlockSpec((BN, LD), blk), pl.BlockSpec((BN, LD), blk),
                pl.BlockSpec((BN, 1), blk),
                pl.BlockSpec((BN, LD), blk), pl.BlockSpec((BN, LD), blk),
                pl.BlockSpec((LD, LD), cst), pl.BlockSpec((LD, LD), cst),
                pl.BlockSpec((1, LD), cst),
                pl.BlockSpec((LD, LD), cst), pl.BlockSpec((1, LD), cst),
                pl.BlockSpec((LD, LD), cst), pl.BlockSpec((1, LD), cst)]
    if want_skip:
        in_specs.append(pl.BlockSpec((BN, LD), blk))
    if want_y:
        in_specs.append(pl.BlockSpec((BN, 1), blk))
    if want_div:
        in_specs.append(pl.BlockSpec((BN, 1), blk))
    out_specs = [pl.BlockSpec((BN, LD), blk)]
    out_shape = [jax.ShapeDtypeStruct((NP, LD), F32)]
    if want_y:
        out_specs.append(pl.BlockSpec((BN, LD), blk))
        out_shape.append(jax.ShapeDtypeStruct((NP, LD), F32))
    if want_div:
        out_specs.append(pl.BlockSpec((BN, LD), blk))
        out_shape.append(jax.ShapeDtypeStruct((NP, LD), F32))
    return pl.pallas_call(
        body,
        grid=(NP // BN,),
        in_specs=in_specs,
        out_specs=tuple(out_specs),
        out_shape=tuple(out_shape),
    )


def _prep_gmp(p):
    w0, b0 = p["edge"][0]
    w1, b1 = p["edge"][1]
    w2, b2 = p["edge"][2]
    wp8 = jnp.concatenate([w0[256:259], jnp.zeros((5, 128), F32)], axis=0)
    wn0, bn0 = p["node"][0]
    wn1, bn1 = p["node"][1]
    wn2, bn2 = p["node"][2]
    return dict(wha=w0[:128], whb=w0[128:256], wp8=wp8,
                w0d=w0[259:260], b0=b0[None, :],
                w1=w1, b1=b1[None, :], w2=w2, b2=b2[None, :],
                wn0a=wn0[:128], wn0b=wn0[128:], bn0=bn0[None, :],
                wn1=wn1, bn1=bn1[None, :], wn2=wn2, bn2=bn2[None, :])


def _pad_rows(x, np_rows):
    return jnp.pad(x, ((0, np_rows - x.shape[0]), (0, 0)))


def _gmp_core(wp, ha, hb, sh, pos8, srcP, dstP, EP, NP,
              zeros128, want_y=False, want_div=False, skip=None,
              nwv=None, sdiv=None):
    a_t, b_t = _tc_ab(NP)(ha, hb, pos8, sh,
                          wp["wha"], wp["whb"], wp["wp8"])
    posx, posy, posz = pos8[:, 0], pos8[:, 1], pos8[:, 2]
    gs, gd, d2 = _sc_gather2(EP, NP)(a_t, b_t, posx, posy, posz, srcP, dstP)
    msg = _tc_edge(EP)(gs, gd, d2[:, None], wp["w0d"], wp["b0"],
                       wp["w1"], wp["b1"], wp["w2"], wp["b2"])
    ag0, ag1 = _sc_scatter_linear(EP, NP)(msg, dstP, zeros128)
    args = [ha, hb, sh, ag0, ag1, wp["wn0a"], wp["wn0b"], wp["bn0"],
            wp["wn1"], wp["bn1"], wp["wn2"], wp["bn2"]]
    if skip is not None:
        args.append(skip)
    if want_y:
        args.append(nwv)
    if want_div:
        args.append(sdiv)
    return _tc_node(NP, want_y, want_div, skip is not None)(*args)


def kernel(h, pos, params, m_ids_0, m_ids_1, m_gs_0, m_gs_1, m_gs_2):
    m_ids = [m_ids_0, m_ids_1]
    m_gs = [m_gs_0, m_gs_1, m_gs_2]
    NS = [h.shape[0], m_ids_0.shape[0], m_ids_1.shape[0]]
    NP = [_rup(n + 8, 256) for n in NS]
    EP = [_rup(g.shape[1], NWORK * CH * 8) for g in m_gs]

    srcP, dstP, srcP2, dstP2 = [], [], [], []
    for l in range(3):
        g = m_gs[l]
        padv = jnp.full((EP[l] - g.shape[1],), NS[l], I32)
        s1d = jnp.concatenate([g[0], padv])
        d1d = jnp.concatenate([g[1], padv])
        srcP.append(s1d)
        dstP.append(d1d)
        srcP2.append(s1d.reshape(EP[l] // CH, CH))
        dstP2.append(d1d.reshape(EP[l] // CH, CH))

    zeros = {n: jnp.zeros((n // NSUB, LD), F32) for n in set(NP)}

    wps = {"down": [_prep_gmp(p) for p in params["down"]],
           "up": [_prep_gmp(p) for p in params["up"]],
           "bottom": _prep_gmp(params["bottom"])}

    onesv = [jnp.ones((n, 1), F32) for n in NP]
    z128 = [jnp.zeros((n, LD), F32) for n in NP]

    ha, hb = _pad_rows(h, NP[0]), z128[0]
    pos8 = _pad_rows(jnp.pad(pos, ((0, 0), (0, 5))), NP[0])
    sh = onesv[0]
    w = jnp.ones((NP[0], 1), F32)

    down_outs, down_pos8, nws, wins = [], [], [], []
    for i in range(2):
        NPi, EPi = NP[i], EP[i]
        d0, d1 = _sc_degree(EPi, NPi)(srcP2[i])
        deg = jnp.maximum((d0 + d1)[:, None], 1.0)
        nwv = w / deg
        hout, Y = _gmp_core(wps["down"][i], ha, hb, sh, pos8,
                            srcP[i], dstP[i], EPi, NPi, zeros[NPi],
                            want_y=True, nwv=nwv)
        down_outs.append(hout)
        down_pos8.append(pos8)
        nws.append(nwv)
        wins.append(w)
        nv = nwv[:, 0]
        aux = _sc_auxpool(EPi, NPi)(nv, nv * pos8[:, 0], nv * pos8[:, 1],
                                    nv * pos8[:, 2], srcP2[i], dstP2[i])
        qn = aux[0] + aux[1]
        qx = aux[2] + aux[3]
        qy = aux[4] + aux[5]
        qz = aux[6] + aux[7]
        p0, p1 = _sc_scatter_gather(EPi, NPi)(Y, srcP[i], dstP[i], zeros[NPi])
        midP = jnp.pad(m_ids[i], (0, NP[i + 1] - NS[i + 1]))
        x2h, awv, pxv, pyv, pzv = _sc_select(NP[i + 1], NPi)(
            p0, p1, qn, qx, qy, qz, midP)
        aw2 = awv[:, None] + 1e-12
        inv_aw = 1.0 / aw2
        ha, hb = x2h, z128[i + 1]
        sh = inv_aw
        pos8 = jnp.pad(jnp.stack([pxv, pyv, pzv], axis=1),
                       ((0, 0), (0, 5))) * inv_aw
        w = aw2

    hbot, hdiv = _gmp_core(wps["bottom"], ha, hb, sh, pos8,
                           srcP[2], dstP[2], EP[2], NP[2], zeros[NP[2]],
                           want_div=True, sdiv=1.0 / w)

    hcur = hbot
    for i in range(2):
        up = 1 - i
        NPf, NPc, EPu = NP[up], NP[up + 1], EP[up]
        nc = NS[up + 1]
        ncp = _rup(nc, CH)
        ids = jnp.pad(m_ids[up], (0, ncp - nc), constant_values=-8)
        ids_next = jnp.pad(m_ids[up][1:], (0, ncp - nc + 1),
                           constant_values=-9)
        rowid = lax.broadcasted_iota(I32, (NPc, 1), 0)
        zc = jnp.where(rowid < nc, hdiv, 0.0)
        jb = _sc_translate(EPu, NPf, ncp, nc)(ids, ids_next, dstP[up])
        c0, c1 = _sc_scatter_gather(EPu, NPf)(zc, jb, srcP[up], zeros[NPf])
        want_div = (i == 0)
        outs = _gmp_core(wps["up"][i], c0, c1, nws[up],
                         down_pos8[up], srcP[up], dstP[up], EPu, NPf,
                         zeros[NPf], want_div=want_div, skip=down_outs[up],
                         sdiv=(1.0 / wins[up]) if want_div else None)
        if want_div:
            hcur, hdiv = outs
        else:
            (hcur,) = outs

    return hcur[:NS[0]]
